# Initial kernel scaffold; baseline (speedup 1.0000x reference)
import jax
import jax.numpy as jnp
from jax.experimental import pallas as pl


def kernel(cate_seq_x, cont_seq_x, r_tab, c_tab, rc_tab, W1, b1, ln_g, ln_b, W2, b2):
    raise NotImplementedError("write your pallas kernel here")


# speedup vs baseline: 7.2089x; 7.2089x over previous
import jax
import jax.numpy as jnp
from jax.experimental import pallas as pl
from jax.experimental.pallas import tpu as pltpu

S = 80
EPS = 1e-5
TM = 8192


def _fused_kernel(cate_ref, cont_ref, r_ref, c_ref, rc_ref, w1_ref, b1_ref,
                  lng_ref, lnb_ref, w2_ref, b2_ref, out_ref):
    w1 = w1_ref[...]
    w_eff = jnp.concatenate(
        [r_ref[...] @ w1[0:2], c_ref[...] @ w1[2:4], rc_ref[...] @ w1[4:8],
         w1[8:]], axis=0)

    p = cate_ref[...]
    tm = p.shape[0]
    ir = jax.lax.bitwise_and(p, 3)
    ic = jax.lax.bitwise_and(jax.lax.shift_right_logical(p, 2), 3)
    irc = jax.lax.shift_right_logical(p, 4)
    i3 = jax.lax.broadcasted_iota(jnp.int32, (tm, 3), 1)
    i9 = jax.lax.broadcasted_iota(jnp.int32, (tm, 9), 1)
    oh_r = (i3 == ir).astype(jnp.float32)
    oh_c = (i3 == ic).astype(jnp.float32)
    oh_rc = (i9 == irc).astype(jnp.float32)
    x = jnp.concatenate([oh_r, oh_c, oh_rc, cont_ref[...]], axis=1)
    h = jax.lax.dot_general(x, w_eff, (((1,), (0,)), ((), ())),
                            preferred_element_type=jnp.float32)
    h = h + b1_ref[0]
    mu = jnp.mean(h, axis=-1, keepdims=True)
    d = h - mu
    var = jnp.mean(d * d, axis=-1, keepdims=True)
    h = d * jax.lax.rsqrt(var + EPS) * lng_ref[0] + lnb_ref[0]
    h = jnp.maximum(h, 0.0)
    out_ref[...] = jnp.sum(h * w2_ref[0], axis=-1, keepdims=True) + b2_ref[0, 0]


def kernel(cate_seq_x, cont_seq_x, r_tab, c_tab, rc_tab, W1, b1, ln_g, ln_b,
           W2, b2):
    B = cont_seq_x.shape[0]
    M = B * S
    tm = TM if M % TM == 0 else M
    cate = cate_seq_x.astype(jnp.int32)
    packed = (cate[:, :, 0] + (cate[:, :, 1] << 2)
              + (cate[:, :, 2] << 4)).reshape(M, 1)
    cont = cont_seq_x.reshape(M, 13)
    grid = (M // tm,)
    rep = lambda i: (0, 0)
    out = pl.pallas_call(
        _fused_kernel,
        grid=grid,
        in_specs=[
            pl.BlockSpec((tm, 1), lambda i: (i, 0)),
            pl.BlockSpec((tm, 13), lambda i: (i, 0)),
            pl.BlockSpec((3, 2), rep),
            pl.BlockSpec((3, 2), rep),
            pl.BlockSpec((9, 4), rep),
            pl.BlockSpec((21, 100), rep),
            pl.BlockSpec((1, 100), rep),
            pl.BlockSpec((1, 100), rep),
            pl.BlockSpec((1, 100), rep),
            pl.BlockSpec((1, 100), rep),
            pl.BlockSpec((1, 1), rep),
        ],
        out_specs=pl.BlockSpec((tm, 1), lambda i: (i, 0)),
        out_shape=jax.ShapeDtypeStruct((M, 1), jnp.float32),
        compiler_params=pltpu.CompilerParams(
            dimension_semantics=("arbitrary",)),
    )(packed, cont, r_tab, c_tab, rc_tab, W1,
      b1.reshape(1, 100), ln_g.reshape(1, 100), ln_b.reshape(1, 100),
      W2.reshape(1, 100), b2.reshape(1, 1))
    return out.reshape(B, S)
